# Initial kernel scaffold; baseline (speedup 1.0000x reference)
import jax
import jax.numpy as jnp
from jax.experimental import pallas as pl


def kernel(x, edge_index, sequences, W1, b1, W2, b2, w_ih, w_hh, b_ih, b_hh, fc_W, fc_b):
    raise NotImplementedError("write your pallas kernel here")


# speedup vs baseline: 1.3733x; 1.3733x over previous
import jax
import jax.numpy as jnp
from jax.experimental import pallas as pl
from jax.experimental.pallas import tpu as pltpu

_B = 64
_T = 50
_H = 128
_RH = 128
_C = 10


def _gru_fc_kernel(seq_ref, w_ih_ref, w_hh_ref, b_ih_ref, b_hh_ref,
                   fc_w_ref, fc_b_ref, out_ref, gi_ref):
    gi_ref[:, :] = jax.lax.dot_general(
        seq_ref[:, :], w_ih_ref[:, :], (((1,), (1,)), ((), ())),
        preferred_element_type=jnp.float32) + b_ih_ref[:, :]

    w_hh = w_hh_ref[:, :]
    b_hh = b_hh_ref[:, :]

    def step(t, h):
        gi = gi_ref[pl.ds(t * _B, _B), :]
        gh = jax.lax.dot_general(
            h, w_hh, (((1,), (1,)), ((), ())),
            preferred_element_type=jnp.float32) + b_hh
        r = jax.nn.sigmoid(gi[:, :_RH] + gh[:, :_RH])
        z = jax.nn.sigmoid(gi[:, _RH:2 * _RH] + gh[:, _RH:2 * _RH])
        n = jnp.tanh(gi[:, 2 * _RH:] + r * gh[:, 2 * _RH:])
        return (1.0 - z) * n + z * h

    h = jax.lax.fori_loop(0, _T, step, jnp.zeros((_B, _RH), jnp.float32))

    out_ref[:, :] = jax.lax.dot_general(
        h, fc_w_ref[:, :], (((1,), (1,)), ((), ())),
        preferred_element_type=jnp.float32) + fc_b_ref[:, :]


def kernel(x, edge_index, sequences, W1, b1, W2, b2,
           w_ih, w_hh, b_ih, b_hh, fc_W, fc_b):
    seq2d = jnp.swapaxes(sequences, 0, 1).reshape(_T * _B, _H)
    return pl.pallas_call(
        _gru_fc_kernel,
        out_shape=jax.ShapeDtypeStruct((_B, _C), jnp.float32),
        scratch_shapes=[pltpu.VMEM((_T * _B, 3 * _RH), jnp.float32)],
    )(seq2d, w_ih, w_hh, b_ih.reshape(1, -1), b_hh.reshape(1, -1),
      fc_W, fc_b.reshape(1, -1))


# speedup vs baseline: 1.5543x; 1.1318x over previous
import jax
import jax.numpy as jnp
from jax.experimental import pallas as pl

_B = 64
_T = 50
_H = 128
_RH = 128
_C = 10


def _dot_t(a, b):
    return jax.lax.dot_general(a, b, (((1,), (1,)), ((), ())),
                               preferred_element_type=jnp.float32)


def _gru_fc_kernel(seq_ref, w_ih_ref, w_hh_ref, brzn_ref, bhn_ref,
                   fc_w_ref, fc_b_ref, out_ref):
    w_ih = w_ih_ref[:, :]
    w_hh = w_hh_ref[:, :]
    brzn = brzn_ref[:, :]
    bhn = bhn_ref[:, :]

    h = jnp.zeros((_B, _RH), jnp.float32)
    for t in range(_T):
        x_t = seq_ref[:, t * _H:(t + 1) * _H]
        g = _dot_t(x_t, w_ih) + brzn
        gh = _dot_t(h, w_hh)
        r = jax.nn.sigmoid(g[:, :_RH] + gh[:, :_RH])
        z = jax.nn.sigmoid(g[:, _RH:2 * _RH] + gh[:, _RH:2 * _RH])
        n = jnp.tanh(g[:, 2 * _RH:] + r * (gh[:, 2 * _RH:] + bhn))
        h = n + z * (h - n)

    out_ref[:, :] = _dot_t(h, fc_w_ref[:, :]) + fc_b_ref[:, :]


def kernel(x, edge_index, sequences, W1, b1, W2, b2,
           w_ih, w_hh, b_ih, b_hh, fc_W, fc_b):
    seqflat = sequences.reshape(_B, _T * _H)
    brzn = (b_ih + jnp.concatenate(
        [b_hh[:2 * _RH], jnp.zeros((_RH,), jnp.float32)])).reshape(1, -1)
    bhn = b_hh[2 * _RH:].reshape(1, -1)
    return pl.pallas_call(
        _gru_fc_kernel,
        out_shape=jax.ShapeDtypeStruct((_B, _C), jnp.float32),
    )(seqflat, w_ih, w_hh, brzn, bhn, fc_W, fc_b.reshape(1, -1))


# speedup vs baseline: 1.5580x; 1.0023x over previous
import jax
import jax.numpy as jnp
from jax.experimental import pallas as pl

_B = 64
_T = 50
_H = 128
_RH = 128
_C = 10


def _dot_t(a, b):
    return jax.lax.dot_general(a, b, (((1,), (1,)), ((), ())),
                               preferred_element_type=jnp.float32)


def _gru_fc_kernel(seq_ref, w_ih_ref, w_hh_ref, brzn_ref, bhn_ref,
                   fc_w_ref, fc_b_ref, out_ref):
    w_ih = w_ih_ref[:, :]
    w_hh = w_hh_ref[:, :]
    brzn = brzn_ref[:, :]
    bhn = bhn_ref[:, :]

    h = jnp.zeros((_B, _RH), jnp.float32)
    for t in range(_T):
        x_t = seq_ref[:, t * _H:(t + 1) * _H]
        g = _dot_t(x_t, w_ih) + brzn
        gh = _dot_t(h, w_hh)
        r = 0.5 + 0.5 * jnp.tanh(0.5 * (g[:, :_RH] + gh[:, :_RH]))
        z = 0.5 + 0.5 * jnp.tanh(0.5 * (g[:, _RH:2 * _RH] + gh[:, _RH:2 * _RH]))
        n = jnp.tanh(g[:, 2 * _RH:] + r * (gh[:, 2 * _RH:] + bhn))
        h = n + z * (h - n)

    out_ref[:, :] = _dot_t(h, fc_w_ref[:, :]) + fc_b_ref[:, :]


def kernel(x, edge_index, sequences, W1, b1, W2, b2,
           w_ih, w_hh, b_ih, b_hh, fc_W, fc_b):
    seqflat = sequences.reshape(_B, _T * _H)
    brzn = (b_ih + jnp.concatenate(
        [b_hh[:2 * _RH], jnp.zeros((_RH,), jnp.float32)])).reshape(1, -1)
    bhn = b_hh[2 * _RH:].reshape(1, -1)
    return pl.pallas_call(
        _gru_fc_kernel,
        out_shape=jax.ShapeDtypeStruct((_B, _C), jnp.float32),
    )(seqflat, w_ih, w_hh, brzn, bhn, fc_W, fc_b.reshape(1, -1))
